# baseline (speedup 1.0000x reference)
import functools

import jax
import jax.numpy as jnp
from jax import lax
from jax.experimental import pallas as pl
from jax.experimental.pallas import tpu as pltpu
from jax.experimental.pallas import tpu_sc as plsc

NC = 2
NS = 16
NW = NC * NS
CHUNK = 128
LANES = 16

HID = 128


def _matmul_bias_kernel(x_ref, w_ref, b_ref, o_ref):
    o_ref[...] = (
        jnp.dot(x_ref[...], w_ref[...], preferred_element_type=jnp.float32)
        + b_ref[...]
    )


def _mlp_kernel(h_ref, p0_ref, p1_ref, w1_ref, b1_ref, w2_ref, b2_ref, o_ref):
    pre = h_ref[...] + p0_ref[...] + p1_ref[...]
    t = jnp.dot(pre, w1_ref[...], preferred_element_type=jnp.float32) + b1_ref[...]
    t = jnp.maximum(t, 0.0)
    o_ref[...] = (
        jnp.dot(t, w2_ref[...], preferred_element_type=jnp.float32) + b2_ref[...]
    )


def _make_sc_agg(n_pad, chunks):
    rows_per_tile = n_pad // NS
    zcopies = rows_per_tile // CHUNK
    assert rows_per_tile % CHUNK == 0 and chunks % 4 == 0 and chunks >= 8

    mesh = plsc.VectorSubcoreMesh(core_axis_name="c", subcore_axis_name="s")

    @functools.partial(
        pl.kernel,
        out_type=jax.ShapeDtypeStruct((NC, n_pad, HID), jnp.float32),
        mesh=mesh,
        scratch_types=[
            pltpu.VMEM((4, 2, CHUNK), jnp.int32),
            pltpu.VMEM((2, CHUNK, HID), jnp.float32),
            pltpu.VMEM_SHARED((n_pad, HID), jnp.float32),
            [pltpu.SemaphoreType.DMA] * 2,
            [pltpu.SemaphoreType.DMA] * 2,
            [pltpu.SemaphoreType.DMA] * 4,
        ],
    )
    def sc_agg(h0_hbm, sd_hbm, out_hbm, sd_v, rows_v, agg_sh,
               sem_g, sem_s, sem_i):
        cid = lax.axis_index("c")
        sid = lax.axis_index("s")
        w = cid * NS + sid

        @pl.loop(0, CHUNK)
        def _zero_rows(i):
            for j in range(HID // LANES):
                rows_v[0, i, pl.ds(j * LANES, LANES)] = jnp.zeros(
                    (LANES,), jnp.float32)

        base_row = sid * rows_per_tile
        for r in range(zcopies):
            pltpu.sync_copy(
                rows_v.at[0],
                agg_sh.at[pl.ds(base_row + r * CHUNK, CHUNK)])

        def fire_idx(j, ib):
            pltpu.async_copy(sd_hbm.at[w, j], sd_v.at[ib], sem_i[ib])

        def wait_idx(j, ib):
            pltpu.make_async_copy(
                sd_hbm.at[w, j], sd_v.at[ib], sem_i[ib]).wait()

        def fire_gather(rb, ib):
            pltpu.async_copy(
                h0_hbm.at[sd_v.at[ib, 0]], rows_v.at[rb], sem_g[rb])

        def wait_gather(rb, ib):
            pltpu.make_async_copy(
                h0_hbm.at[sd_v.at[ib, 0]], rows_v.at[rb], sem_g[rb]).wait()

        def fire_scat(rb, ib):
            pltpu.async_copy(
                rows_v.at[rb], agg_sh.at[sd_v.at[ib, 1]], sem_s[rb], add=True)

        def wait_scat(rb, ib):
            pltpu.make_async_copy(
                rows_v.at[rb], agg_sh.at[sd_v.at[ib, 1]], sem_s[rb]).wait()


        pltpu.sync_copy(sd_hbm.at[w, 0], sd_v.at[0])
        fire_idx(1, 1)
        fire_gather(0, 0)
        plsc.subcore_barrier()

        wait_gather(0, 0)
        fire_scat(0, 0)
        wait_idx(1, 1)
        fire_gather(1, 1)
        fire_idx(2, 2)
        wait_gather(1, 1)
        fire_scat(1, 1)
        wait_idx(2, 2)
        wait_scat(0, 0)
        fire_gather(0, 2)
        fire_idx(3, 3)

        @pl.loop(2, chunks - 2, step=4)
        def _body(j):
            for k in range(4):
                rb = k % 2
                ib = (2 + k) % 4
                wait_gather(rb, ib)
                fire_scat(rb, ib)
                wait_idx(j + k + 1, (3 + k) % 4)
                wait_scat(1 - rb, (1 + k) % 4)
                fire_gather(1 - rb, (3 + k) % 4)
                fire_idx(j + k + 2, (k) % 4)

        wait_gather(0, 2)
        fire_scat(0, 2)
        wait_idx(chunks - 1, 3)
        wait_scat(1, 1)
        fire_gather(1, 3)
        wait_gather(1, 3)
        fire_scat(1, 3)
        wait_scat(0, 2)
        wait_scat(1, 3)

        plsc.subcore_barrier()
        pltpu.sync_copy(
            agg_sh.at[pl.ds(base_row, rows_per_tile)],
            out_hbm.at[cid, pl.ds(base_row, rows_per_tile)])

    return sc_agg


def kernel(x, edge_index, W0, b0, W1, b1, W2, b2):
    n, d_in = x.shape
    e = edge_index.shape[1]
    hid = W0.shape[1]

    row_blk = 1000
    grid = (n + row_blk - 1) // row_blk
    b0r = b0.reshape(1, hid)
    h0 = pl.pallas_call(
        _matmul_bias_kernel,
        grid=(grid,),
        in_specs=[
            pl.BlockSpec((row_blk, d_in), lambda i: (i, 0)),
            pl.BlockSpec((d_in, hid), lambda i: (0, 0)),
            pl.BlockSpec((1, hid), lambda i: (0, 0)),
        ],
        out_specs=pl.BlockSpec((row_blk, hid), lambda i: (i, 0)),
        out_shape=jax.ShapeDtypeStruct((n, hid), jnp.float32),
    )(x, W0, b0r)

    per_worker = -(-e // NW)
    chunks = -(-per_worker // CHUNK)
    if chunks % 2:
        chunks += 1
    e_pad = NW * chunks * CHUNK
    n_pad = -(-(n + 1) // (NS * CHUNK)) * (NS * CHUNK)

    src = edge_index[0].astype(jnp.int32)
    dst = edge_index[1].astype(jnp.int32)
    pad = e_pad - e
    src_p = jnp.concatenate([src, jnp.zeros((pad,), jnp.int32)])
    dst_p = jnp.concatenate([dst, jnp.full((pad,), n, jnp.int32)])
    sd = jnp.stack(
        [src_p.reshape(NW, chunks, CHUNK), dst_p.reshape(NW, chunks, CHUNK)],
        axis=2)

    partials = _make_sc_agg(n_pad, chunks)(h0, sd)
    p0 = partials[0, :n]
    p1 = partials[1, :n]

    b1r = b1.reshape(1, hid)
    b2r = b2.reshape(1, hid)
    l1 = pl.pallas_call(
        _mlp_kernel,
        grid=(grid,),
        in_specs=[
            pl.BlockSpec((row_blk, hid), lambda i: (i, 0)),
            pl.BlockSpec((row_blk, hid), lambda i: (i, 0)),
            pl.BlockSpec((row_blk, hid), lambda i: (i, 0)),
            pl.BlockSpec((hid, hid), lambda i: (0, 0)),
            pl.BlockSpec((1, hid), lambda i: (0, 0)),
            pl.BlockSpec((hid, hid), lambda i: (0, 0)),
            pl.BlockSpec((1, hid), lambda i: (0, 0)),
        ],
        out_specs=pl.BlockSpec((row_blk, hid), lambda i: (i, 0)),
        out_shape=jax.ShapeDtypeStruct((n, hid), jnp.float32),
    )(h0, p0, p1, W1, b1r, W2, b2r)

    return (l1, h0)


# speedup vs baseline: 2.9497x; 2.9497x over previous
import functools

import jax
import jax.numpy as jnp
from jax import lax
from jax.experimental import pallas as pl
from jax.experimental.pallas import tpu as pltpu
from jax.experimental.pallas import tpu_sc as plsc

NC = 2
NS = 16
NW = NC * NS
CHUNK = 128
LANES = 16

HID = 128


def _matmul_bias_kernel(x_ref, w_ref, b_ref, o_ref):
    o_ref[...] = (
        jnp.dot(x_ref[...], w_ref[...], preferred_element_type=jnp.float32)
        + b_ref[...]
    )


def _mlp_kernel(h_ref, p0_ref, p1_ref, w1_ref, b1_ref, w2_ref, b2_ref, o_ref):
    pre = h_ref[...] + p0_ref[...] + p1_ref[...]
    t = jnp.dot(pre, w1_ref[...], preferred_element_type=jnp.float32) + b1_ref[...]
    t = jnp.maximum(t, 0.0)
    o_ref[...] = (
        jnp.dot(t, w2_ref[...], preferred_element_type=jnp.float32) + b2_ref[...]
    )


def _make_sc_agg(n_pad, chunks):
    rows_per_tile = n_pad // NS
    zcopies = rows_per_tile // CHUNK
    assert rows_per_tile % CHUNK == 0 and chunks % 4 == 0 and chunks >= 8

    mesh = plsc.VectorSubcoreMesh(core_axis_name="c", subcore_axis_name="s")

    @functools.partial(
        pl.kernel,
        out_type=jax.ShapeDtypeStruct((NC, n_pad, HID), jnp.float32),
        mesh=mesh,
        scratch_types=[
            pltpu.VMEM((4, 2, CHUNK), jnp.int32),
            pltpu.VMEM((2, CHUNK, HID), jnp.float32),
            pltpu.VMEM_SHARED((n_pad, HID), jnp.float32),
            [pltpu.SemaphoreType.DMA] * 2,
            [pltpu.SemaphoreType.DMA] * 2,
            [pltpu.SemaphoreType.DMA] * 4,
        ],
    )
    def sc_agg(h0_hbm, sd_hbm, out_hbm, sd_v, rows_v, agg_sh,
               sem_g, sem_s, sem_i):
        cid = lax.axis_index("c")
        sid = lax.axis_index("s")
        w = cid * NS + sid

        @pl.loop(0, CHUNK)
        def _zero_rows(i):
            for j in range(HID // LANES):
                rows_v[0, i, pl.ds(j * LANES, LANES)] = jnp.zeros(
                    (LANES,), jnp.float32)

        base_row = sid * rows_per_tile
        for r in range(zcopies):
            pltpu.sync_copy(
                rows_v.at[0],
                agg_sh.at[pl.ds(base_row + r * CHUNK, CHUNK)])

        def fire_idx(j, ib):
            pltpu.async_copy(sd_hbm.at[w, j], sd_v.at[ib], sem_i[ib])

        def wait_idx(j, ib):
            pltpu.make_async_copy(
                sd_hbm.at[w, j], sd_v.at[ib], sem_i[ib]).wait()

        def fire_gather(rb, ib):
            pltpu.async_copy(
                h0_hbm.at[sd_v.at[ib, 0]], rows_v.at[rb], sem_g[rb])

        def wait_gather(rb, ib):
            pltpu.make_async_copy(
                h0_hbm.at[sd_v.at[ib, 0]], rows_v.at[rb], sem_g[rb]).wait()

        def fire_scat(rb, ib):
            pltpu.async_copy(
                rows_v.at[rb], agg_sh.at[sd_v.at[ib, 1]], sem_s[rb], add=True)

        def wait_scat(rb, ib):
            pltpu.make_async_copy(
                rows_v.at[rb], agg_sh.at[sd_v.at[ib, 1]], sem_s[rb]).wait()


        pltpu.sync_copy(sd_hbm.at[w, 0], sd_v.at[0])
        fire_idx(1, 1)
        fire_gather(0, 0)
        plsc.subcore_barrier()

        wait_gather(0, 0)
        fire_scat(0, 0)
        wait_idx(1, 1)
        fire_gather(1, 1)
        fire_idx(2, 2)
        wait_gather(1, 1)
        fire_scat(1, 1)
        wait_idx(2, 2)
        wait_scat(0, 0)
        fire_gather(0, 2)
        fire_idx(3, 3)

        @pl.loop(2, chunks - 2, step=4)
        def _body(j):
            for k in range(4):
                rb = k % 2
                ib = (2 + k) % 4
                wait_gather(rb, ib)
                fire_scat(rb, ib)
                wait_idx(j + k + 1, (3 + k) % 4)
                wait_scat(1 - rb, (1 + k) % 4)
                fire_gather(1 - rb, (3 + k) % 4)
                fire_idx(j + k + 2, (k) % 4)

        wait_gather(0, 2)
        fire_scat(0, 2)
        wait_idx(chunks - 1, 3)
        wait_scat(1, 1)
        fire_gather(1, 3)
        wait_gather(1, 3)
        fire_scat(1, 3)
        wait_scat(0, 2)
        wait_scat(1, 3)

        plsc.subcore_barrier()
        pltpu.sync_copy(
            agg_sh.at[pl.ds(base_row, rows_per_tile)],
            out_hbm.at[cid, pl.ds(base_row, rows_per_tile)])

    return sc_agg


def kernel(x, edge_index, W0, b0, W1, b1, W2, b2):
    n, d_in = x.shape
    e = edge_index.shape[1]
    hid = W0.shape[1]

    row_blk = 1000
    grid = (n + row_blk - 1) // row_blk
    b0r = b0.reshape(1, hid)
    h0 = pl.pallas_call(
        _matmul_bias_kernel,
        grid=(grid,),
        in_specs=[
            pl.BlockSpec((row_blk, d_in), lambda i: (i, 0)),
            pl.BlockSpec((d_in, hid), lambda i: (0, 0)),
            pl.BlockSpec((1, hid), lambda i: (0, 0)),
        ],
        out_specs=pl.BlockSpec((row_blk, hid), lambda i: (i, 0)),
        out_shape=jax.ShapeDtypeStruct((n, hid), jnp.float32),
    )(x, W0, b0r)

    per_worker = -(-e // NW)
    chunks = -(-per_worker // CHUNK)
    chunks += (-chunks) % 4
    e_pad = NW * chunks * CHUNK
    n_pad = -(-(n + 1) // (NS * CHUNK)) * (NS * CHUNK)

    src = edge_index[0].astype(jnp.int32)
    dst = edge_index[1].astype(jnp.int32)
    pad = e_pad - e
    src_p = jnp.concatenate([src, (jnp.arange(pad, dtype=jnp.int32) * 37) % n])
    dst_p = jnp.concatenate([dst, jnp.full((pad,), n, jnp.int32)])
    sd = jnp.stack(
        [src_p.reshape(NW, chunks, CHUNK), dst_p.reshape(NW, chunks, CHUNK)],
        axis=2)

    partials = _make_sc_agg(n_pad, chunks)(h0, sd)
    p0 = partials[0, :n]
    p1 = partials[1, :n]

    b1r = b1.reshape(1, hid)
    b2r = b2.reshape(1, hid)
    l1 = pl.pallas_call(
        _mlp_kernel,
        grid=(grid,),
        in_specs=[
            pl.BlockSpec((row_blk, hid), lambda i: (i, 0)),
            pl.BlockSpec((row_blk, hid), lambda i: (i, 0)),
            pl.BlockSpec((row_blk, hid), lambda i: (i, 0)),
            pl.BlockSpec((hid, hid), lambda i: (0, 0)),
            pl.BlockSpec((1, hid), lambda i: (0, 0)),
            pl.BlockSpec((hid, hid), lambda i: (0, 0)),
            pl.BlockSpec((1, hid), lambda i: (0, 0)),
        ],
        out_specs=pl.BlockSpec((row_blk, hid), lambda i: (i, 0)),
        out_shape=jax.ShapeDtypeStruct((n, hid), jnp.float32),
    )(h0, p0, p1, W1, b1r, W2, b2r)

    return (l1, h0)


# speedup vs baseline: 3.0719x; 1.0414x over previous
import functools

import jax
import jax.numpy as jnp
from jax import lax
from jax.experimental import pallas as pl
from jax.experimental.pallas import tpu as pltpu
from jax.experimental.pallas import tpu_sc as plsc

NC = 2
NS = 16
NW = NC * NS
CHUNK = 128
LANES = 16

HID = 128


def _matmul_bias_kernel(x_ref, w_ref, b_ref, o_ref):
    o_ref[...] = (
        jnp.dot(x_ref[...], w_ref[...], preferred_element_type=jnp.float32)
        + b_ref[...]
    )


def _mlp_kernel(h_ref, p_ref, w1_ref, b1_ref, w2_ref, b2_ref, o_ref):
    pre = h_ref[...] + p_ref[0] + p_ref[1]
    t = jnp.dot(pre, w1_ref[...], preferred_element_type=jnp.float32) + b1_ref[...]
    t = jnp.maximum(t, 0.0)
    o_ref[...] = (
        jnp.dot(t, w2_ref[...], preferred_element_type=jnp.float32) + b2_ref[...]
    )


def _make_sc_agg(n_pad, chunks):
    rows_per_tile = n_pad // NS
    zcopies = rows_per_tile // CHUNK
    assert rows_per_tile % CHUNK == 0 and chunks % 4 == 0 and chunks >= 8

    mesh = plsc.VectorSubcoreMesh(core_axis_name="c", subcore_axis_name="s")

    @functools.partial(
        pl.kernel,
        out_type=jax.ShapeDtypeStruct((NC, n_pad, HID), jnp.float32),
        mesh=mesh,
        scratch_types=[
            pltpu.VMEM((4, 2, CHUNK), jnp.int32),
            pltpu.VMEM((2, CHUNK, HID), jnp.float32),
            pltpu.VMEM_SHARED((n_pad, HID), jnp.float32),
            [pltpu.SemaphoreType.DMA] * 2,
            [pltpu.SemaphoreType.DMA] * 2,
            [pltpu.SemaphoreType.DMA] * 4,
        ],
    )
    def sc_agg(h0_hbm, sd_hbm, out_hbm, sd_v, rows_v, agg_sh,
               sem_g, sem_s, sem_i):
        cid = lax.axis_index("c")
        sid = lax.axis_index("s")
        w = cid * NS + sid

        @pl.loop(0, CHUNK)
        def _zero_rows(i):
            for j in range(HID // LANES):
                rows_v[0, i, pl.ds(j * LANES, LANES)] = jnp.zeros(
                    (LANES,), jnp.float32)

        base_row = sid * rows_per_tile
        for r in range(zcopies):
            pltpu.sync_copy(
                rows_v.at[0],
                agg_sh.at[pl.ds(base_row + r * CHUNK, CHUNK)])

        def fire_idx(j, ib):
            pltpu.async_copy(sd_hbm.at[w, j], sd_v.at[ib], sem_i[ib])

        def wait_idx(j, ib):
            pltpu.make_async_copy(
                sd_hbm.at[w, j], sd_v.at[ib], sem_i[ib]).wait()

        def fire_gather(rb, ib):
            pltpu.async_copy(
                h0_hbm.at[sd_v.at[ib, 0]], rows_v.at[rb], sem_g[rb])

        def wait_gather(rb, ib):
            pltpu.make_async_copy(
                h0_hbm.at[sd_v.at[ib, 0]], rows_v.at[rb], sem_g[rb]).wait()

        def fire_scat(rb, ib):
            pltpu.async_copy(
                rows_v.at[rb], agg_sh.at[sd_v.at[ib, 1]], sem_s[rb], add=True)

        def wait_scat(rb, ib):
            pltpu.make_async_copy(
                rows_v.at[rb], agg_sh.at[sd_v.at[ib, 1]], sem_s[rb]).wait()


        pltpu.sync_copy(sd_hbm.at[w, 0], sd_v.at[0])
        fire_idx(1, 1)
        fire_gather(0, 0)
        plsc.subcore_barrier()

        wait_gather(0, 0)
        fire_scat(0, 0)
        wait_idx(1, 1)
        fire_gather(1, 1)
        fire_idx(2, 2)
        wait_gather(1, 1)
        fire_scat(1, 1)
        wait_idx(2, 2)
        wait_scat(0, 0)
        fire_gather(0, 2)
        fire_idx(3, 3)

        @pl.loop(2, chunks - 2, step=4)
        def _body(j):
            for k in range(4):
                rb = k % 2
                ib = (2 + k) % 4
                wait_gather(rb, ib)
                fire_scat(rb, ib)
                wait_idx(j + k + 1, (3 + k) % 4)
                wait_scat(1 - rb, (1 + k) % 4)
                fire_gather(1 - rb, (3 + k) % 4)
                fire_idx(j + k + 2, (k) % 4)

        wait_gather(0, 2)
        fire_scat(0, 2)
        wait_idx(chunks - 1, 3)
        wait_scat(1, 1)
        fire_gather(1, 3)
        wait_gather(1, 3)
        fire_scat(1, 3)
        wait_scat(0, 2)
        wait_scat(1, 3)

        plsc.subcore_barrier()
        pltpu.sync_copy(
            agg_sh.at[pl.ds(base_row, rows_per_tile)],
            out_hbm.at[cid, pl.ds(base_row, rows_per_tile)])

    return sc_agg


def kernel(x, edge_index, W0, b0, W1, b1, W2, b2):
    n, d_in = x.shape
    e = edge_index.shape[1]
    hid = W0.shape[1]

    row_blk = 1000
    grid = (n + row_blk - 1) // row_blk
    b0r = b0.reshape(1, hid)
    h0 = pl.pallas_call(
        _matmul_bias_kernel,
        grid=(grid,),
        in_specs=[
            pl.BlockSpec((row_blk, d_in), lambda i: (i, 0)),
            pl.BlockSpec((d_in, hid), lambda i: (0, 0)),
            pl.BlockSpec((1, hid), lambda i: (0, 0)),
        ],
        out_specs=pl.BlockSpec((row_blk, hid), lambda i: (i, 0)),
        out_shape=jax.ShapeDtypeStruct((n, hid), jnp.float32),
    )(x, W0, b0r)

    per_worker = -(-e // NW)
    chunks = -(-per_worker // CHUNK)
    chunks += (-chunks) % 4
    e_pad = NW * chunks * CHUNK
    n_pad = -(-(n + 1) // (NS * CHUNK)) * (NS * CHUNK)

    src = edge_index[0].astype(jnp.int32)
    dst = edge_index[1].astype(jnp.int32)
    pad = e_pad - e
    src_p = jnp.concatenate([src, (jnp.arange(pad, dtype=jnp.int32) * 37) % n])
    dst_p = jnp.concatenate([dst, jnp.full((pad,), n, jnp.int32)])
    sd = jnp.stack(
        [src_p.reshape(NW, chunks, CHUNK), dst_p.reshape(NW, chunks, CHUNK)],
        axis=2)

    partials = _make_sc_agg(n_pad, chunks)(h0, sd)

    b1r = b1.reshape(1, hid)
    b2r = b2.reshape(1, hid)
    l1 = pl.pallas_call(
        _mlp_kernel,
        grid=(grid,),
        in_specs=[
            pl.BlockSpec((row_blk, hid), lambda i: (i, 0)),
            pl.BlockSpec((NC, row_blk, hid), lambda i: (0, i, 0)),
            pl.BlockSpec((hid, hid), lambda i: (0, 0)),
            pl.BlockSpec((1, hid), lambda i: (0, 0)),
            pl.BlockSpec((hid, hid), lambda i: (0, 0)),
            pl.BlockSpec((1, hid), lambda i: (0, 0)),
        ],
        out_specs=pl.BlockSpec((row_blk, hid), lambda i: (i, 0)),
        out_shape=jax.ShapeDtypeStruct((n, hid), jnp.float32),
    )(h0, partials, W1, b1r, W2, b2r)

    return (l1, h0)
